# baseline (speedup 1.0000x reference)
import functools
import math

import jax
import jax.numpy as jnp
from jax import lax
from jax.experimental import pallas as pl
from jax.experimental.pallas import tpu as pltpu
from jax.experimental.pallas import tpu_sc as plsc

_NC = 2
_NS = 16
_NW = _NC * _NS
_CHUNK = 256
_NBUF = 2
_LANES = 16


def _sc_gather_scale(table, idx, scale):
    (n,) = idx.shape
    _, d = table.shape
    npw = n // _NW
    cpw = npw // _CHUNK
    assert n % _NW == 0 and npw % _CHUNK == 0 and cpw % _NBUF == 0

    mesh = plsc.VectorSubcoreMesh(core_axis_name="c", subcore_axis_name="s")

    @functools.partial(
        pl.kernel,
        out_type=jax.ShapeDtypeStruct((n, d), table.dtype),
        mesh=mesh,
        scratch_types=[
            pltpu.VMEM((npw,), jnp.int32),
            pltpu.VMEM((_NBUF, _CHUNK, d), table.dtype),
            pltpu.SemaphoreType.DMA((_NBUF,)),
            pltpu.SemaphoreType.DMA((_NBUF,)),
        ],
    )
    def k(table_hbm, idx_hbm, out_hbm, idx_v, rows_v, gsem, ssem):
        wid = lax.axis_index("s") * _NC + lax.axis_index("c")
        r0 = wid * npw
        pltpu.sync_copy(idx_hbm.at[pl.ds(r0, npw)], idx_v)
        for b in range(_NBUF):
            pltpu.async_copy(
                table_hbm.at[idx_v.at[pl.ds(b * _CHUNK, _CHUNK)]],
                rows_v.at[b],
                gsem.at[b],
            )

        @pl.loop(0, cpw, step=_NBUF)
        def _(g0):
            for b in range(_NBUF):
                g = g0 + b
                pltpu.make_async_copy(
                    table_hbm.at[idx_v.at[pl.ds(g * _CHUNK, _CHUNK)]],
                    rows_v.at[b],
                    gsem.at[b],
                ).wait()

                @pl.loop(0, _CHUNK, unroll=8)
                def _(r):
                    for c in range(d // _LANES):
                        sl = pl.ds(c * _LANES, _LANES)
                        rows_v[b, r, sl] = rows_v[b, r, sl] * scale

                row0 = r0 + g * _CHUNK
                pltpu.async_copy(
                    rows_v.at[b], out_hbm.at[pl.ds(row0, _CHUNK)], ssem.at[b]
                )
            for b in range(_NBUF):
                g = g0 + b
                row0 = r0 + g * _CHUNK
                pltpu.make_async_copy(
                    rows_v.at[b], out_hbm.at[pl.ds(row0, _CHUNK)], ssem.at[b]
                ).wait()

                @pl.when(g + _NBUF < cpw)
                def _():
                    pltpu.async_copy(
                        table_hbm.at[
                            idx_v.at[pl.ds((g + _NBUF) * _CHUNK, _CHUNK)]
                        ],
                        rows_v.at[b],
                        gsem.at[b],
                    )

    return k(table, idx)


def kernel(tokens, table):
    bsz, seq = tokens.shape
    _, d = table.shape
    n = bsz * seq
    scale = math.sqrt(d)
    idx = tokens.reshape(n).astype(jnp.int32)
    out = _sc_gather_scale(table, idx, scale)
    return out.reshape(bsz, seq, d)


# speedup vs baseline: 1.0086x; 1.0086x over previous
import functools
import math

import jax
import jax.numpy as jnp
from jax import lax
from jax.experimental import pallas as pl
from jax.experimental.pallas import tpu as pltpu
from jax.experimental.pallas import tpu_sc as plsc

_NC = 2
_NS = 16
_NW = _NC * _NS
_CHUNK = 128
_NBUF = 5
_LANES = 16


def _sc_gather_scale(table, idx2d, scale):
    nchunks, chunk = idx2d.shape
    _, d = table.shape
    n = nchunks * chunk
    cpw = nchunks // _NW
    assert nchunks % _NW == 0 and cpw % _NBUF == 0 and d % _LANES == 0

    mesh = plsc.VectorSubcoreMesh(core_axis_name="c", subcore_axis_name="s")

    @functools.partial(
        pl.kernel,
        out_type=jax.ShapeDtypeStruct((n, d), table.dtype),
        mesh=mesh,
        scratch_types=[
            pltpu.VMEM((cpw, chunk), jnp.int32),
            pltpu.VMEM((_NBUF, chunk, d), table.dtype),
            pltpu.SemaphoreType.DMA((_NBUF,)),
            pltpu.SemaphoreType.DMA((_NBUF,)),
        ],
    )
    def k(table_hbm, idx_hbm, out_hbm, idx_v, rows_v, gsem, ssem):
        wid = lax.axis_index("s") * _NC + lax.axis_index("c")
        c0 = wid * cpw
        pltpu.sync_copy(idx_hbm.at[pl.ds(c0, cpw)], idx_v)
        for b in range(_NBUF):
            pltpu.async_copy(table_hbm.at[idx_v.at[b]], rows_v.at[b], gsem.at[b])

        @pl.loop(0, cpw, step=_NBUF)
        def _(g0):
            for b in range(_NBUF):
                g = g0 + b
                pltpu.make_async_copy(
                    table_hbm.at[idx_v.at[g]], rows_v.at[b], gsem.at[b]
                ).wait()

                @pl.loop(0, chunk, unroll=8)
                def _(r):
                    for c in range(d // _LANES):
                        sl = pl.ds(c * _LANES, _LANES)
                        rows_v[b, r, sl] = rows_v[b, r, sl] * scale

                row0 = (c0 + g) * chunk
                pltpu.async_copy(
                    rows_v.at[b], out_hbm.at[pl.ds(row0, chunk)], ssem.at[b]
                )
            for b in range(_NBUF):
                g = g0 + b
                row0 = (c0 + g) * chunk
                pltpu.make_async_copy(
                    rows_v.at[b], out_hbm.at[pl.ds(row0, chunk)], ssem.at[b]
                ).wait()

                @pl.when(g + _NBUF < cpw)
                def _():
                    pltpu.async_copy(
                        table_hbm.at[idx_v.at[g + _NBUF]], rows_v.at[b], gsem.at[b]
                    )

    return k(table, idx2d)


def kernel(tokens, table):
    bsz, seq = tokens.shape
    _, d = table.shape
    n = bsz * seq
    assert n % (_NW * _CHUNK) == 0
    scale = math.sqrt(d)
    idx2d = tokens.reshape(n // _CHUNK, _CHUNK).astype(jnp.int32)
    out = _sc_gather_scale(table, idx2d, scale)
    return out.reshape(bsz, seq, d)
